# baseline (speedup 1.0000x reference)
import functools

import jax
import jax.numpy as jnp
from jax import lax
from jax.experimental import pallas as pl
from jax.experimental.pallas import tpu as pltpu, tpu_sc as plsc

NUM_NODES = 10000
NUM_EDGES = 320000
HIDDEN = 128

EDGES_A = 192000
EDGES_B = NUM_EDGES - EDGES_A

NC = 2
NS = 16
NBUF = 3
CHUNK = 128
ROWS_MAIN = 640
ROWS_TAIL = NUM_NODES - (NS - 1) * ROWS_MAIN

EDGE_BLOCK = 16000
NODE_BLOCK = 1000


def _edge_body(rbft_ref, x_ref, wt_ref, t_ref):
    s = lax.dot_general(
        rbft_ref[...], wt_ref[...], (((0,), (0,)), ((), ())),
        preferred_element_type=jnp.float32,
    )
    t_ref[...] = s * x_ref[...]


def _edge_stage(rbft, x, wt, start_edge, n_edges):
    grid = (n_edges // EDGE_BLOCK,)
    off = start_edge // EDGE_BLOCK
    return pl.pallas_call(
        _edge_body,
        grid=grid,
        in_specs=[
            pl.BlockSpec((rbft.shape[0], EDGE_BLOCK), lambda i: (0, i + off)),
            pl.BlockSpec((EDGE_BLOCK, HIDDEN), lambda i: (i + off, 0)),
            pl.BlockSpec(wt.shape, lambda i: (0, 0)),
        ],
        out_specs=pl.BlockSpec((EDGE_BLOCK, HIDDEN), lambda i: (i, 0)),
        out_shape=jax.ShapeDtypeStruct((n_edges, HIDDEN), jnp.float32),
    )(rbft, x, wt)


def _scatter_body(start_edge, edges_per_core, edges_per_sub, num_chunks, tail,
                  t_hbm, i_hbm, z_hbm, out_hbm,
                  idx0, rows0, idx1, rows1, idx2, rows2, idx_t,
                  sem_i0, sem_r0, sem_i1, sem_r1, sem_i2, sem_r2, acc_sh):
    c = lax.axis_index("c")
    s = lax.axis_index("s")

    @pl.when(s < NS - 1)
    def _():
        pltpu.sync_copy(
            z_hbm.at[pl.ds(s * ROWS_MAIN, ROWS_MAIN)],
            acc_sh.at[pl.ds(s * ROWS_MAIN, ROWS_MAIN)],
        )

    @pl.when(s == NS - 1)
    def _():
        pltpu.sync_copy(
            z_hbm.at[pl.ds((NS - 1) * ROWS_MAIN, ROWS_TAIL)],
            acc_sh.at[pl.ds((NS - 1) * ROWS_MAIN, ROWS_TAIL)],
        )

    plsc.subcore_barrier()

    base0 = c * edges_per_core + s * edges_per_sub

    pltpu.sync_copy(i_hbm.at[pl.ds(start_edge + base0, tail)], idx_t)
    pltpu.sync_copy(t_hbm.at[pl.ds(base0, tail)], rows0.at[pl.ds(0, tail)])
    pltpu.sync_copy(rows0.at[pl.ds(0, tail)], acc_sh.at[idx_t], add=True)

    bufs = ((idx0, rows0, sem_i0, sem_r0),
            (idx1, rows1, sem_i1, sem_r1),
            (idx2, rows2, sem_i2, sem_r2))

    def start_load(k, idx_v, rows_v, sem_i, sem_r):
        base = base0 + tail + k * CHUNK
        pltpu.async_copy(i_hbm.at[pl.ds(start_edge + base, CHUNK)], idx_v, sem_i)
        pltpu.async_copy(t_hbm.at[pl.ds(base, CHUNK)], rows_v, sem_r)

    def wait_load(k, idx_v, rows_v, sem_i, sem_r):
        base = base0 + tail + k * CHUNK
        pltpu.make_async_copy(
            i_hbm.at[pl.ds(start_edge + base, CHUNK)], idx_v, sem_i).wait()
        pltpu.make_async_copy(t_hbm.at[pl.ds(base, CHUNK)], rows_v, sem_r).wait()

    start_load(0, *bufs[0])
    start_load(1, *bufs[1])

    def step(k, b):
        idx_v, rows_v, sem_i, sem_r = bufs[b]
        wait_load(k, idx_v, rows_v, sem_i, sem_r)

        if isinstance(k, int):
            if k + 2 < num_chunks:
                start_load(k + 2, *bufs[(b + 2) % NBUF])
        else:
            @pl.when(k + 2 < num_chunks)
            def _():
                start_load(k + 2, *bufs[(b + 2) % NBUF])

        pltpu.sync_copy(rows_v, acc_sh.at[idx_v], add=True)

    def group(p, _):
        for b in range(NBUF):
            step(NBUF * p + b, b)
        return _

    full_groups = num_chunks // NBUF
    lax.fori_loop(0, full_groups, group, None)
    for k in range(full_groups * NBUF, num_chunks):
        step(k, k % NBUF)

    plsc.subcore_barrier()

    @pl.when(s < NS - 1)
    def _():
        pltpu.sync_copy(
            acc_sh.at[pl.ds(s * ROWS_MAIN, ROWS_MAIN)],
            out_hbm.at[c, pl.ds(s * ROWS_MAIN, ROWS_MAIN)],
        )

    @pl.when(s == NS - 1)
    def _():
        pltpu.sync_copy(
            acc_sh.at[pl.ds((NS - 1) * ROWS_MAIN, ROWS_TAIL)],
            out_hbm.at[c, pl.ds((NS - 1) * ROWS_MAIN, ROWS_TAIL)],
        )


def _make_scatter_stage(start_edge, n_edges):
    edges_per_core = n_edges // NC
    edges_per_sub = edges_per_core // NS
    num_chunks = edges_per_sub // CHUNK
    tail = edges_per_sub - num_chunks * CHUNK
    assert tail % 8 == 0 and tail <= CHUNK and start_edge % 8 == 0
    return pl.kernel(
        functools.partial(_scatter_body, start_edge, edges_per_core,
                          edges_per_sub, num_chunks, tail),
        out_type=jax.ShapeDtypeStruct((NC, NUM_NODES, HIDDEN), jnp.float32),
        mesh=plsc.VectorSubcoreMesh(core_axis_name="c", subcore_axis_name="s"),
        scratch_types=[
            pltpu.VMEM((CHUNK,), jnp.int32),
            pltpu.VMEM((CHUNK, HIDDEN), jnp.float32),
            pltpu.VMEM((CHUNK,), jnp.int32),
            pltpu.VMEM((CHUNK, HIDDEN), jnp.float32),
            pltpu.VMEM((CHUNK,), jnp.int32),
            pltpu.VMEM((CHUNK, HIDDEN), jnp.float32),
            pltpu.VMEM((tail,), jnp.int32),
            pltpu.SemaphoreType.DMA,
            pltpu.SemaphoreType.DMA,
            pltpu.SemaphoreType.DMA,
            pltpu.SemaphoreType.DMA,
            pltpu.SemaphoreType.DMA,
            pltpu.SemaphoreType.DMA,
            pltpu.VMEM_SHARED((NUM_NODES, HIDDEN), jnp.float32),
        ],
    )


_scatter_stage_a = _make_scatter_stage(0, EDGES_A)
_scatter_stage_b = _make_scatter_stage(EDGES_A, EDGES_B)


def _bdot(a, b):
    return lax.dot_general(
        a.astype(jnp.bfloat16), b.astype(jnp.bfloat16),
        (((1,), (1,)), ((), ())),
        preferred_element_type=jnp.float32,
    )


def _mlp_body(pa_ref, pb_ref, wup_ref, wl_ref, bl_ref, wout_ref, out_ref):
    xt = (pa_ref[0] + pa_ref[1]) + (pb_ref[0] + pb_ref[1])
    h = _bdot(xt, wup_ref[...])
    for l in range(wl_ref.shape[0]):
        z = _bdot(h, wl_ref[l]) + bl_ref[l][None, :]
        h = z * jax.nn.sigmoid(z)
    out_ref[...] = lax.dot_general(
        h, wout_ref[...], (((1,), (1,)), ((), ())),
        preferred_element_type=jnp.float32,
    )


def _mlp_stage(parts_a, parts_b, w_up, w_layers, b_layers, w_out):
    grid = (NUM_NODES // NODE_BLOCK,)
    return pl.pallas_call(
        _mlp_body,
        grid=grid,
        in_specs=[
            pl.BlockSpec((NC, NODE_BLOCK, HIDDEN), lambda j: (0, j, 0)),
            pl.BlockSpec((NC, NODE_BLOCK, HIDDEN), lambda j: (0, j, 0)),
            pl.BlockSpec(w_up.shape, lambda j: (0, 0)),
            pl.BlockSpec(w_layers.shape, lambda j: (0, 0, 0)),
            pl.BlockSpec(b_layers.shape, lambda j: (0, 0)),
            pl.BlockSpec(w_out.shape, lambda j: (0, 0)),
        ],
        out_specs=pl.BlockSpec((NODE_BLOCK, w_out.shape[0]), lambda j: (j, 0)),
        out_shape=jax.ShapeDtypeStruct((NUM_NODES, w_out.shape[0]), jnp.float32),
    )(parts_a, parts_b, w_up, w_layers, b_layers, w_out)


def kernel(x, rbf, i, num_nodes, W_rbfs, W_up, W_layers, b_layers, W_out):
    wt = jnp.transpose(W_rbfs[-1])
    zeros = jnp.zeros((NUM_NODES, HIDDEN), jnp.float32)
    rbft = jnp.transpose(rbf)
    t_a = _edge_stage(rbft, x, wt, 0, EDGES_A)
    parts_a = _scatter_stage_a(t_a, i, zeros)
    t_b = _edge_stage(rbft, x, wt, EDGES_A, EDGES_B)
    parts_b = _scatter_stage_b(t_b, i, zeros)
    return _mlp_stage(parts_a, parts_b, W_up, W_layers, b_layers, W_out)


# speedup vs baseline: 1.0097x; 1.0097x over previous
import functools

import jax
import jax.numpy as jnp
from jax import lax
from jax.experimental import pallas as pl
from jax.experimental.pallas import tpu as pltpu, tpu_sc as plsc

NUM_NODES = 10000
NUM_EDGES = 320000
HIDDEN = 128

EDGES_A = 160000
EDGES_B = NUM_EDGES - EDGES_A

NC = 2
NS = 16
NBUF = 3
CHUNK = 128
ROWS_MAIN = 640
ROWS_TAIL = NUM_NODES - (NS - 1) * ROWS_MAIN

EDGE_BLOCK = 16000
NODE_BLOCK = 1000


def _edge_body(rbft_ref, x_ref, wt_ref, t_ref):
    s = lax.dot_general(
        rbft_ref[...], wt_ref[...], (((0,), (0,)), ((), ())),
        preferred_element_type=jnp.float32,
    )
    t_ref[...] = s * x_ref[...]


def _edge_stage(rbft, x, wt, start_edge, n_edges):
    grid = (n_edges // EDGE_BLOCK,)
    off = start_edge // EDGE_BLOCK
    return pl.pallas_call(
        _edge_body,
        grid=grid,
        in_specs=[
            pl.BlockSpec((rbft.shape[0], EDGE_BLOCK), lambda i: (0, i + off)),
            pl.BlockSpec((EDGE_BLOCK, HIDDEN), lambda i: (i + off, 0)),
            pl.BlockSpec(wt.shape, lambda i: (0, 0)),
        ],
        out_specs=pl.BlockSpec((EDGE_BLOCK, HIDDEN), lambda i: (i, 0)),
        out_shape=jax.ShapeDtypeStruct((n_edges, HIDDEN), jnp.float32),
    )(rbft, x, wt)


def _scatter_body(start_edge, edges_per_core, edges_per_sub, num_chunks, tail,
                  t_hbm, i_hbm, z_hbm, out_hbm,
                  idx0, rows0, idx1, rows1, idx2, rows2, idx_t,
                  sem_i0, sem_r0, sem_i1, sem_r1, sem_i2, sem_r2, acc_sh):
    c = lax.axis_index("c")
    s = lax.axis_index("s")

    @pl.when(s < NS - 1)
    def _():
        pltpu.sync_copy(
            z_hbm.at[pl.ds(s * ROWS_MAIN, ROWS_MAIN)],
            acc_sh.at[pl.ds(s * ROWS_MAIN, ROWS_MAIN)],
        )

    @pl.when(s == NS - 1)
    def _():
        pltpu.sync_copy(
            z_hbm.at[pl.ds((NS - 1) * ROWS_MAIN, ROWS_TAIL)],
            acc_sh.at[pl.ds((NS - 1) * ROWS_MAIN, ROWS_TAIL)],
        )

    plsc.subcore_barrier()

    base0 = c * edges_per_core + s * edges_per_sub

    pltpu.sync_copy(i_hbm.at[pl.ds(start_edge + base0, tail)], idx_t)
    pltpu.sync_copy(t_hbm.at[pl.ds(base0, tail)], rows0.at[pl.ds(0, tail)])
    pltpu.sync_copy(rows0.at[pl.ds(0, tail)], acc_sh.at[idx_t], add=True)

    bufs = ((idx0, rows0, sem_i0, sem_r0),
            (idx1, rows1, sem_i1, sem_r1),
            (idx2, rows2, sem_i2, sem_r2))

    def start_load(k, idx_v, rows_v, sem_i, sem_r):
        base = base0 + tail + k * CHUNK
        pltpu.async_copy(i_hbm.at[pl.ds(start_edge + base, CHUNK)], idx_v, sem_i)
        pltpu.async_copy(t_hbm.at[pl.ds(base, CHUNK)], rows_v, sem_r)

    def wait_load(k, idx_v, rows_v, sem_i, sem_r):
        base = base0 + tail + k * CHUNK
        pltpu.make_async_copy(
            i_hbm.at[pl.ds(start_edge + base, CHUNK)], idx_v, sem_i).wait()
        pltpu.make_async_copy(t_hbm.at[pl.ds(base, CHUNK)], rows_v, sem_r).wait()

    start_load(0, *bufs[0])
    start_load(1, *bufs[1])

    def step(k, b):
        idx_v, rows_v, sem_i, sem_r = bufs[b]
        wait_load(k, idx_v, rows_v, sem_i, sem_r)

        if isinstance(k, int):
            if k + 2 < num_chunks:
                start_load(k + 2, *bufs[(b + 2) % NBUF])
        else:
            @pl.when(k + 2 < num_chunks)
            def _():
                start_load(k + 2, *bufs[(b + 2) % NBUF])

        pltpu.sync_copy(rows_v, acc_sh.at[idx_v], add=True)

    def group(p, _):
        for b in range(NBUF):
            step(NBUF * p + b, b)
        return _

    full_groups = num_chunks // NBUF
    lax.fori_loop(0, full_groups, group, None)
    for k in range(full_groups * NBUF, num_chunks):
        step(k, k % NBUF)

    plsc.subcore_barrier()

    @pl.when(s < NS - 1)
    def _():
        pltpu.sync_copy(
            acc_sh.at[pl.ds(s * ROWS_MAIN, ROWS_MAIN)],
            out_hbm.at[c, pl.ds(s * ROWS_MAIN, ROWS_MAIN)],
        )

    @pl.when(s == NS - 1)
    def _():
        pltpu.sync_copy(
            acc_sh.at[pl.ds((NS - 1) * ROWS_MAIN, ROWS_TAIL)],
            out_hbm.at[c, pl.ds((NS - 1) * ROWS_MAIN, ROWS_TAIL)],
        )


def _make_scatter_stage(start_edge, n_edges):
    edges_per_core = n_edges // NC
    edges_per_sub = edges_per_core // NS
    num_chunks = edges_per_sub // CHUNK
    tail = edges_per_sub - num_chunks * CHUNK
    assert tail % 8 == 0 and tail <= CHUNK and start_edge % 8 == 0
    return pl.kernel(
        functools.partial(_scatter_body, start_edge, edges_per_core,
                          edges_per_sub, num_chunks, tail),
        out_type=jax.ShapeDtypeStruct((NC, NUM_NODES, HIDDEN), jnp.float32),
        mesh=plsc.VectorSubcoreMesh(core_axis_name="c", subcore_axis_name="s"),
        scratch_types=[
            pltpu.VMEM((CHUNK,), jnp.int32),
            pltpu.VMEM((CHUNK, HIDDEN), jnp.float32),
            pltpu.VMEM((CHUNK,), jnp.int32),
            pltpu.VMEM((CHUNK, HIDDEN), jnp.float32),
            pltpu.VMEM((CHUNK,), jnp.int32),
            pltpu.VMEM((CHUNK, HIDDEN), jnp.float32),
            pltpu.VMEM((tail,), jnp.int32),
            pltpu.SemaphoreType.DMA,
            pltpu.SemaphoreType.DMA,
            pltpu.SemaphoreType.DMA,
            pltpu.SemaphoreType.DMA,
            pltpu.SemaphoreType.DMA,
            pltpu.SemaphoreType.DMA,
            pltpu.VMEM_SHARED((NUM_NODES, HIDDEN), jnp.float32),
        ],
    )


_scatter_stage_a = _make_scatter_stage(0, EDGES_A)
_scatter_stage_b = _make_scatter_stage(EDGES_A, EDGES_B)


def _bdot(a, b):
    return lax.dot_general(
        a.astype(jnp.bfloat16), b.astype(jnp.bfloat16),
        (((1,), (1,)), ((), ())),
        preferred_element_type=jnp.float32,
    )


def _mlp_body(pa_ref, pb_ref, wup_ref, wl_ref, bl_ref, wout_ref, out_ref):
    xt = (pa_ref[0] + pa_ref[1]) + (pb_ref[0] + pb_ref[1])
    h = _bdot(xt, wup_ref[...])
    for l in range(wl_ref.shape[0]):
        z = _bdot(h, wl_ref[l]) + bl_ref[l][None, :]
        h = z * jax.nn.sigmoid(z)
    out_ref[...] = lax.dot_general(
        h, wout_ref[...], (((1,), (1,)), ((), ())),
        preferred_element_type=jnp.float32,
    )


def _mlp_stage(parts_a, parts_b, w_up, w_layers, b_layers, w_out):
    grid = (NUM_NODES // NODE_BLOCK,)
    return pl.pallas_call(
        _mlp_body,
        grid=grid,
        in_specs=[
            pl.BlockSpec((NC, NODE_BLOCK, HIDDEN), lambda j: (0, j, 0)),
            pl.BlockSpec((NC, NODE_BLOCK, HIDDEN), lambda j: (0, j, 0)),
            pl.BlockSpec(w_up.shape, lambda j: (0, 0)),
            pl.BlockSpec(w_layers.shape, lambda j: (0, 0, 0)),
            pl.BlockSpec(b_layers.shape, lambda j: (0, 0)),
            pl.BlockSpec(w_out.shape, lambda j: (0, 0)),
        ],
        out_specs=pl.BlockSpec((NODE_BLOCK, w_out.shape[0]), lambda j: (j, 0)),
        out_shape=jax.ShapeDtypeStruct((NUM_NODES, w_out.shape[0]), jnp.float32),
    )(parts_a, parts_b, w_up, w_layers, b_layers, w_out)


def kernel(x, rbf, i, num_nodes, W_rbfs, W_up, W_layers, b_layers, W_out):
    wt = jnp.transpose(W_rbfs[-1])
    zeros = jnp.zeros((NUM_NODES, HIDDEN), jnp.float32)
    rbft = jnp.transpose(rbf)
    t_a = _edge_stage(rbft, x, wt, 0, EDGES_A)
    parts_a = _scatter_stage_a(t_a, i, zeros)
    t_b = _edge_stage(rbft, x, wt, EDGES_A, EDGES_B)
    parts_b = _scatter_stage_b(t_b, i, zeros)
    return _mlp_stage(parts_a, parts_b, W_up, W_layers, b_layers, W_out)
